# baseline (speedup 1.0000x reference)
import functools

import jax
import jax.numpy as jnp
from jax import lax
from jax.experimental import pallas as pl
from jax.experimental.pallas import tpu as pltpu
from jax.experimental.pallas import tpu_sc as plsc

B, N, M, K = 4, 16384, 1024, 64
L = 16
NCHUNK = N // L
UNROLL = 16
RSTRIDE = K + L


def _ball_query_body(xyz_hbm, ct_hbm, r2_hbm, out_hbm,
                     xs, ys, zs, x2s, cb, r2s, ob, ob2):
    nc = 2
    wid = lax.axis_index("s") * nc + lax.axis_index("c")
    cpw = (B * M) // 32
    wpb = M // cpw
    b = wid // wpb
    m0 = (wid % wpb) * cpw

    pltpu.sync_copy(xyz_hbm.at[pl.ds((b * 3 + 0) * N, N)], xs)
    pltpu.sync_copy(xyz_hbm.at[pl.ds((b * 3 + 1) * N, N)], ys)
    pltpu.sync_copy(xyz_hbm.at[pl.ds((b * 3 + 2) * N, N)], zs)
    pltpu.sync_copy(ct_hbm.at[pl.ds((b * 3 + 0) * M + m0, cpw)],
                    cb.at[pl.ds(0 * cpw, cpw)])
    pltpu.sync_copy(ct_hbm.at[pl.ds((b * 3 + 1) * M + m0, cpw)],
                    cb.at[pl.ds(1 * cpw, cpw)])
    pltpu.sync_copy(ct_hbm.at[pl.ds((b * 3 + 2) * M + m0, cpw)],
                    cb.at[pl.ds(2 * cpw, cpw)])
    pltpu.sync_copy(r2_hbm, r2s)

    r2v = r2s[...]
    iota = lax.iota(jnp.int32, L)
    zeros = jnp.zeros((L,), jnp.int32)

    def bf16_round(v):
        u = plsc.bitcast(v, jnp.int32)
        lsb = lax.shift_right_logical(u, 16) & 1
        u = (u + (lsb + 0x7FFF)) & jnp.int32(-65536)
        return plsc.bitcast(u, jnp.float32)

    def pre(j, _):
        base = j * L
        px = xs[pl.ds(base, L)]
        py = ys[pl.ds(base, L)]
        pz = zs[pl.ds(base, L)]
        x2s[pl.ds(base, L)] = (px * px + py * py) + pz * pz
        xs[pl.ds(base, L)] = bf16_round(px)
        ys[pl.ds(base, L)] = bf16_round(py)
        zs[pl.ds(base, L)] = bf16_round(pz)
        return 0

    lax.fori_loop(0, NCHUNK, pre, 0)

    kvec = jnp.full((L,), K, jnp.int32)

    def per_group(g, _):
        cx16 = cb[pl.ds(0 * cpw + g * L, L)]
        cy16 = cb[pl.ds(1 * cpw + g * L, L)]
        cz16 = cb[pl.ds(2 * cpw + g * L, L)]
        c216 = (cx16 * cx16 + cy16 * cy16) + cz16 * cz16
        cxb16 = bf16_round(cx16)
        cxb16 = cxb16 + cxb16
        cyb16 = bf16_round(cy16)
        cyb16 = cyb16 + cyb16
        czb16 = bf16_round(cz16)
        czb16 = czb16 + czb16
        for lane in range(L // 2):
            row_a = (g * L + lane) * RSTRIDE
            row_b = row_a + (L // 2) * RSTRIDE
            for v in range(K // L):
                ob[pl.ds(row_a + v * L, L)] = zeros
                ob[pl.ds(row_b + v * L, L)] = zeros

            cxa = jnp.full((L,), cxb16[lane], jnp.float32)
            cya = jnp.full((L,), cyb16[lane], jnp.float32)
            cza = jnp.full((L,), czb16[lane], jnp.float32)
            c2a = jnp.full((L,), c216[lane], jnp.float32)
            cxc = jnp.full((L,), cxb16[lane + L // 2], jnp.float32)
            cyc = jnp.full((L,), cyb16[lane + L // 2], jnp.float32)
            czc = jnp.full((L,), czb16[lane + L // 2], jnp.float32)
            c2c = jnp.full((L,), c216[lane + L // 2], jnp.float32)

            def cond(carry):
                j, live, cnta, cntb = carry
                return jnp.logical_and(j < NCHUNK, live)

            def body(carry, row_a=row_a, row_b=row_b, cxa=cxa, cya=cya,
                     cza=cza, c2a=c2a, cxc=cxc, cyc=cyc, czc=czc, c2c=c2c):
                j, live, cnta, cntb = carry
                live = jnp.logical_or(cnta[0] < K, cntb[0] < K)
                hitsa, hitsb, offsa, offsb, idxs = [], [], [], [], []
                for u in range(UNROLL):
                    base = (j + u) * L
                    px = xs[pl.ds(base, L)]
                    py = ys[pl.ds(base, L)]
                    pz = zs[pl.ds(base, L)]
                    x2 = x2s[pl.ds(base, L)]
                    idxs.append(iota + base)
                    crossa = (cxa * px + cya * py) + cza * pz
                    crossb = (cxc * px + cyc * py) + czc * pz
                    hita = ((c2a + x2) - crossa) < r2v
                    hitb = ((c2c + x2) - crossb) < r2v
                    hitsa.append(hita)
                    hitsb.append(hitb)
                    offsa.append(cnta[0])
                    offsb.append(cntb[0])
                    cnta = jnp.minimum(
                        cnta + plsc.all_reduce_population_count(hita), kvec)
                    cntb = jnp.minimum(
                        cntb + plsc.all_reduce_population_count(hitb), kvec)
                for u in range(UNROLL):
                    plsc.store_compressed(ob.at[pl.ds(row_a + offsa[u], L)],
                                          idxs[u], mask=hitsa[u])
                    plsc.store_compressed(ob.at[pl.ds(row_b + offsb[u], L)],
                                          idxs[u], mask=hitsb[u])
                return j + UNROLL, live, cnta, cntb

            lax.while_loop(cond, body,
                           (jnp.int32(0), jnp.bool_(True), zeros, zeros))
        for lane in range(L):
            src = (g * L + lane) * RSTRIDE
            dst = (g * L + lane) * K
            for v in range(K // L):
                ob2[pl.ds(dst + v * L, L)] = ob[pl.ds(src + v * L, L)]
        return 0

    lax.fori_loop(0, cpw // L, per_group, 0)
    pltpu.sync_copy(ob2.at[pl.ds(0, cpw * K)],
                    out_hbm.at[pl.ds((b * M + m0) * K, cpw * K)])


def kernel(xyz, center_xyz, max_radius, sample_num):
    xyz_t = jnp.transpose(xyz, (0, 2, 1)).reshape(-1)
    ct_t = jnp.transpose(center_xyz, (0, 2, 1)).reshape(-1)
    r2 = jnp.asarray(max_radius, jnp.float32) ** 2
    r2v = jnp.broadcast_to(r2, (L,))

    cpw = (B * M) // 32
    mesh = plsc.VectorSubcoreMesh(core_axis_name="c", subcore_axis_name="s")
    run = functools.partial(
        pl.kernel,
        mesh=mesh,
        out_type=jax.ShapeDtypeStruct((B * M * K,), jnp.int32),
        scratch_types=[
            pltpu.VMEM((N,), jnp.float32),
            pltpu.VMEM((N,), jnp.float32),
            pltpu.VMEM((N,), jnp.float32),
            pltpu.VMEM((N,), jnp.float32),
            pltpu.VMEM((3 * cpw,), jnp.float32),
            pltpu.VMEM((L,), jnp.float32),
            pltpu.VMEM((cpw * RSTRIDE,), jnp.int32),
            pltpu.VMEM((cpw * K,), jnp.int32),
        ],
        compiler_params=pltpu.CompilerParams(needs_layout_passes=False),
    )(_ball_query_body)
    idx = run(xyz_t, ct_t, r2v).reshape(B, M, K)
    col = lax.broadcasted_iota(jnp.int32, (1, 1, K), 2)
    return jnp.where(col < jnp.asarray(sample_num, jnp.int32), idx, 0)


# speedup vs baseline: 1.2654x; 1.2654x over previous
import functools

import jax
import jax.numpy as jnp
from jax import lax
from jax.experimental import pallas as pl
from jax.experimental.pallas import tpu as pltpu
from jax.experimental.pallas import tpu_sc as plsc

B, N, M, K = 4, 16384, 1024, 64
L = 16
NCHUNK = N // L
UNROLL = 8
GROUPS = M // L
NS = 16
WPB = 8
RSTRIDE = K + L


def _ball_query_body(xyz_hbm, ct_hbm, r2_hbm, out_hbm,
                     xs, ys, zs, x2s, cb, r2s, ob, ob2, wcnt, sem):
    cid = lax.axis_index("c")
    sid = lax.axis_index("s")
    wid = cid * NS + sid
    b = wid // WPB
    leader = (sid // WPB) * WPB

    pltpu.sync_copy(xyz_hbm.at[pl.ds((b * 3 + 0) * N, N)], xs)
    pltpu.sync_copy(xyz_hbm.at[pl.ds((b * 3 + 1) * N, N)], ys)
    pltpu.sync_copy(xyz_hbm.at[pl.ds((b * 3 + 2) * N, N)], zs)
    pltpu.sync_copy(ct_hbm.at[pl.ds((b * 3 + 0) * M, M)],
                    cb.at[pl.ds(0 * M, M)])
    pltpu.sync_copy(ct_hbm.at[pl.ds((b * 3 + 1) * M, M)],
                    cb.at[pl.ds(1 * M, M)])
    pltpu.sync_copy(ct_hbm.at[pl.ds((b * 3 + 2) * M, M)],
                    cb.at[pl.ds(2 * M, M)])
    pltpu.sync_copy(r2_hbm, r2s)

    r2v = r2s[...]
    iota = lax.iota(jnp.int32, L)
    zeros = jnp.zeros((L,), jnp.int32)

    def bf16_round(v):
        u = plsc.bitcast(v, jnp.int32)
        lsb = lax.shift_right_logical(u, 16) & 1
        u = (u + (lsb + 0x7FFF)) & jnp.int32(-65536)
        return plsc.bitcast(u, jnp.float32)

    def pre(j, _):
        base = j * L
        px = xs[pl.ds(base, L)]
        py = ys[pl.ds(base, L)]
        pz = zs[pl.ds(base, L)]
        x2s[pl.ds(base, L)] = (px * px + py * py) + pz * pz
        xs[pl.ds(base, L)] = bf16_round(px)
        ys[pl.ds(base, L)] = bf16_round(py)
        zs[pl.ds(base, L)] = bf16_round(pz)
        return 0

    lax.fori_loop(0, NCHUNK, pre, 0)

    kvec = jnp.full((L,), K, jnp.int32)

    wcnt[0] = jnp.int32(0)
    plsc.subcore_barrier()

    def per_group(g, slot):
        cx16 = cb[pl.ds(0 * M + g * L, L)]
        cy16 = cb[pl.ds(1 * M + g * L, L)]
        cz16 = cb[pl.ds(2 * M + g * L, L)]
        c216 = (cx16 * cx16 + cy16 * cy16) + cz16 * cz16
        cxb16 = bf16_round(cx16)
        cxb16 = cxb16 + cxb16
        cyb16 = bf16_round(cy16)
        cyb16 = cyb16 + cyb16
        czb16 = bf16_round(cz16)
        czb16 = czb16 + czb16
        for lane in range(L // 2):
            row_a = lane * RSTRIDE
            row_b = row_a + (L // 2) * RSTRIDE
            for v in range(K // L):
                ob[pl.ds(row_a + v * L, L)] = zeros
                ob[pl.ds(row_b + v * L, L)] = zeros

            cxa = jnp.full((L,), cxb16[lane], jnp.float32)
            cya = jnp.full((L,), cyb16[lane], jnp.float32)
            cza = jnp.full((L,), czb16[lane], jnp.float32)
            c2a = jnp.full((L,), c216[lane], jnp.float32)
            cxc = jnp.full((L,), cxb16[lane + L // 2], jnp.float32)
            cyc = jnp.full((L,), cyb16[lane + L // 2], jnp.float32)
            czc = jnp.full((L,), czb16[lane + L // 2], jnp.float32)
            c2c = jnp.full((L,), c216[lane + L // 2], jnp.float32)

            def cond(carry):
                j, live, cnta, cntb = carry
                return jnp.logical_and(j < NCHUNK, live)

            def body(carry, row_a=row_a, row_b=row_b, cxa=cxa, cya=cya,
                     cza=cza, c2a=c2a, cxc=cxc, cyc=cyc, czc=czc, c2c=c2c):
                j, live, cnta, cntb = carry
                live = jnp.logical_or(cnta[0] < K, cntb[0] < K)
                hitsa, hitsb, offsa, offsb, idxs = [], [], [], [], []
                for u in range(UNROLL):
                    base = (j + u) * L
                    px = xs[pl.ds(base, L)]
                    py = ys[pl.ds(base, L)]
                    pz = zs[pl.ds(base, L)]
                    x2 = x2s[pl.ds(base, L)]
                    idxs.append(iota + base)
                    crossa = (cxa * px + cya * py) + cza * pz
                    crossb = (cxc * px + cyc * py) + czc * pz
                    hita = ((c2a + x2) - crossa) < r2v
                    hitb = ((c2c + x2) - crossb) < r2v
                    hitsa.append(hita)
                    hitsb.append(hitb)
                    offsa.append(cnta[0])
                    offsb.append(cntb[0])
                    cnta = jnp.minimum(
                        cnta + plsc.all_reduce_population_count(hita), kvec)
                    cntb = jnp.minimum(
                        cntb + plsc.all_reduce_population_count(hitb), kvec)
                for u in range(UNROLL):
                    plsc.store_compressed(ob.at[pl.ds(row_a + offsa[u], L)],
                                          idxs[u], mask=hitsa[u])
                    plsc.store_compressed(ob.at[pl.ds(row_b + offsb[u], L)],
                                          idxs[u], mask=hitsb[u])
                return j + UNROLL, live, cnta, cntb

            lax.while_loop(cond, body,
                           (jnp.int32(0), jnp.bool_(True), zeros, zeros))
        for lane in range(L):
            src = lane * RSTRIDE
            dst = slot * (L * K) + lane * K
            for v in range(K // L):
                ob2[pl.ds(dst + v * L, L)] = ob[pl.ds(src + v * L, L)]
        pltpu.async_copy(ob2.at[pl.ds(slot * (L * K), L * K)],
                         out_hbm.at[pl.ds((b * M + g * L) * K, L * K)], sem)

    def drain_one(slot):
        pltpu.make_async_copy(
            out_hbm.at[pl.ds(0, L * K)],
            ob2.at[pl.ds(slot * (L * K), L * K)], sem).wait()

    def steal_cond(carry):
        return carry[0] < GROUPS

    def steal_body(carry):
        g, i = carry
        slot = i & 1
        pl.when(i >= 2)(lambda: drain_one(slot))
        per_group(g, slot)
        return plsc.fetch_and_add(wcnt, 1, subcore_id=leader), i + 1

    g0 = plsc.fetch_and_add(wcnt, 1, subcore_id=leader)
    _, ndone = lax.while_loop(steal_cond, steal_body, (g0, jnp.int32(0)))
    pl.when(ndone >= 1)(lambda: drain_one(jnp.int32(0) & 1))
    pl.when(ndone >= 2)(lambda: drain_one(jnp.int32(1)))


def kernel(xyz, center_xyz, max_radius, sample_num):
    xyz_t = jnp.transpose(xyz, (0, 2, 1)).reshape(-1)
    ct_t = jnp.transpose(center_xyz, (0, 2, 1)).reshape(-1)
    r2 = jnp.asarray(max_radius, jnp.float32) ** 2
    r2v = jnp.broadcast_to(r2, (L,))

    cpw = (B * M) // 32
    mesh = plsc.VectorSubcoreMesh(core_axis_name="c", subcore_axis_name="s")
    run = functools.partial(
        pl.kernel,
        mesh=mesh,
        out_type=jax.ShapeDtypeStruct((B * M * K,), jnp.int32),
        scratch_types=[
            pltpu.VMEM((N,), jnp.float32),
            pltpu.VMEM((N,), jnp.float32),
            pltpu.VMEM((N,), jnp.float32),
            pltpu.VMEM((N,), jnp.float32),
            pltpu.VMEM((3 * M,), jnp.float32),
            pltpu.VMEM((L,), jnp.float32),
            pltpu.VMEM((L * RSTRIDE,), jnp.int32),
            pltpu.VMEM((2 * L * K,), jnp.int32),
            pltpu.SMEM((1,), jnp.int32),
            pltpu.SemaphoreType.DMA,
        ],
        compiler_params=pltpu.CompilerParams(needs_layout_passes=False),
    )(_ball_query_body)
    idx = run(xyz_t, ct_t, r2v).reshape(B, M, K)
    col = lax.broadcasted_iota(jnp.int32, (1, 1, K), 2)
    return jnp.where(col < jnp.asarray(sample_num, jnp.int32), idx, 0)
